# baseline (speedup 1.0000x reference)
import functools

import jax
import jax.numpy as jnp
from jax import lax
from jax.experimental import pallas as pl
from jax.experimental.pallas import tpu as pltpu
from jax.experimental.pallas import tpu_sc as plsc

N_DOCS = 1000000
N_RANKS = 20
BATCH = 16384
TOTAL = BATCH * N_RANKS
NW = 32
B_PER_W = TOTAL // NW
VECS = B_PER_W // 16
PAT = 80
PAT_VECS = PAT // 16

_mesh = plsc.VectorSubcoreMesh(core_axis_name="c", subcore_axis_name="s")


def _sigmoid16(v):
    return 1.0 / (1.0 + jnp.exp(-v))


@functools.partial(
    pl.kernel,
    mesh=_mesh,
    out_type=[
        jax.ShapeDtypeStruct((TOTAL,), jnp.float32),
        jax.ShapeDtypeStruct((32,), jnp.float32),
        jax.ShapeDtypeStruct((TOTAL,), jnp.float32),
    ],
    scratch_types=[
        pltpu.VMEM((B_PER_W,), jnp.int32),
        pltpu.VMEM((B_PER_W,), jnp.float32),
        pltpu.VMEM((B_PER_W,), jnp.float32),
        pltpu.VMEM((32,), jnp.float32),
        pltpu.VMEM((PAT,), jnp.float32),
        pltpu.SemaphoreType.DMA,
    ],
)
def _pbm_kernel(x_hbm, rel_hbm, exam_hbm, pat_hbm, y_hbm, exam_out_hbm,
                rel_out_hbm, idx_v, rel_v, y_v, exam_v, pat_v, sem):
    wid = lax.axis_index("s") * 2 + lax.axis_index("c")
    base = wid * B_PER_W

    pltpu.sync_copy(x_hbm.at[pl.ds(base, B_PER_W)], idx_v)
    gather = pltpu.async_copy(rel_hbm.at[idx_v], rel_v, sem)

    pltpu.sync_copy(exam_hbm, exam_v)
    for v in range(2):
        sl = pl.ds(v * 16, 16)
        exam_v[sl] = _sigmoid16(exam_v[sl])
    pltpu.sync_copy(pat_hbm, pat_v)
    for v in range(PAT_VECS):
        sl = pl.ds(v * 16, 16)
        pat_v[sl] = _sigmoid16(pat_v[sl])

    gather.wait()

    def body(j, _):
        sl = pl.ds(j * 16, 16)
        s = _sigmoid16(rel_v[sl])
        rel_v[sl] = s
        y_v[sl] = s * pat_v[pl.ds((j % PAT_VECS) * 16, 16)]
        return _

    lax.fori_loop(0, VECS, body, None)

    pltpu.sync_copy(rel_v, rel_out_hbm.at[pl.ds(base, B_PER_W)])
    pltpu.sync_copy(y_v, y_hbm.at[pl.ds(base, B_PER_W)])

    @pl.when(wid == 0)
    def _():
        pltpu.sync_copy(exam_v, exam_out_hbm)


@jax.jit
def kernel(x, rel_table, exam_table):
    xf = x.reshape(TOTAL)
    relf = rel_table.reshape(N_DOCS)
    examf = exam_table.reshape(N_RANKS)
    exam32 = jnp.pad(examf, (0, 32 - N_RANKS))
    pat80 = jnp.tile(examf, PAT // N_RANKS)
    y, exam_o, rel_o = _pbm_kernel(xf, relf, exam32, pat80)
    return (
        y.reshape(BATCH, N_RANKS),
        exam_o[:N_RANKS],
        rel_o.reshape(BATCH, N_RANKS),
    )


# speedup vs baseline: 1.0383x; 1.0383x over previous
import functools

import jax
import jax.numpy as jnp
from jax import lax
from jax.experimental import pallas as pl
from jax.experimental.pallas import tpu as pltpu
from jax.experimental.pallas import tpu_sc as plsc

N_DOCS = 1000000
N_RANKS = 20
BATCH = 16384
TOTAL = BATCH * N_RANKS
NW = 32
B_PER_W = TOTAL // NW
VECS = B_PER_W // 16
PAT = 80
PAT_VECS = PAT // 16
CHUNKS = 4
CH_ELEMS = B_PER_W // CHUNKS

_mesh = plsc.VectorSubcoreMesh(core_axis_name="c", subcore_axis_name="s")


def _sigmoid16(v):
    return 1.0 / (1.0 + jnp.exp(-v))


@functools.partial(
    pl.kernel,
    mesh=_mesh,
    out_type=[
        jax.ShapeDtypeStruct((TOTAL,), jnp.float32),
        jax.ShapeDtypeStruct((32,), jnp.float32),
        jax.ShapeDtypeStruct((TOTAL,), jnp.float32),
    ],
    scratch_types=[
        pltpu.VMEM((B_PER_W,), jnp.int32),
        pltpu.VMEM((B_PER_W,), jnp.float32),
        pltpu.VMEM((B_PER_W,), jnp.float32),
        pltpu.VMEM((32,), jnp.float32),
        pltpu.VMEM((PAT,), jnp.float32),
        pltpu.SemaphoreType.DMA,
        pltpu.SemaphoreType.DMA,
        pltpu.SemaphoreType.DMA,
        pltpu.SemaphoreType.DMA,
        pltpu.SemaphoreType.DMA,
        pltpu.SemaphoreType.DMA,
    ],
)
def _pbm_kernel(x_hbm, rel_hbm, exam_hbm, pat_hbm, y_hbm, exam_out_hbm,
                rel_out_hbm, idx_v, rel_v, y_v, exam_v, pat_v,
                g0, g1, g2, g3, wsem_r, wsem_y):
    wid = lax.axis_index("s") * 2 + lax.axis_index("c")
    base = wid * B_PER_W
    gsems = (g0, g1, g2, g3)

    pltpu.sync_copy(x_hbm.at[pl.ds(base, B_PER_W)], idx_v)
    gathers = []
    for c in range(CHUNKS):
        sl = pl.ds(c * CH_ELEMS, CH_ELEMS)
        gathers.append(
            pltpu.async_copy(rel_hbm.at[idx_v.at[sl]], rel_v.at[sl], gsems[c]))

    pltpu.sync_copy(exam_hbm, exam_v)
    for v in range(2):
        sl = pl.ds(v * 16, 16)
        exam_v[sl] = _sigmoid16(exam_v[sl])
    pltpu.sync_copy(pat_hbm, pat_v)
    for v in range(PAT_VECS):
        sl = pl.ds(v * 16, 16)
        pat_v[sl] = _sigmoid16(pat_v[sl])
    pats = [pat_v[pl.ds(k * 16, 16)] for k in range(PAT_VECS)]

    writes = []
    for c in range(CHUNKS):
        gathers[c].wait()
        cbase = c * CH_ELEMS

        def body(j, _, cbase=cbase):
            b = cbase + j * PAT
            for k in range(PAT_VECS):
                sl = pl.ds(b + k * 16, 16)
                s = _sigmoid16(rel_v[sl])
                rel_v[sl] = s
                y_v[sl] = s * pats[k]
            return _

        lax.fori_loop(0, CH_ELEMS // PAT, body, None)

        sl = pl.ds(cbase, CH_ELEMS)
        osl = pl.ds(base + cbase, CH_ELEMS)
        writes.append(pltpu.async_copy(rel_v.at[sl], rel_out_hbm.at[osl], wsem_r))
        writes.append(pltpu.async_copy(y_v.at[sl], y_hbm.at[osl], wsem_y))

    @pl.when(wid == 0)
    def _():
        pltpu.sync_copy(exam_v, exam_out_hbm)

    for w in writes:
        w.wait()


@jax.jit
def kernel(x, rel_table, exam_table):
    xf = x.reshape(TOTAL)
    relf = rel_table.reshape(N_DOCS)
    examf = exam_table.reshape(N_RANKS)
    exam32 = jnp.pad(examf, (0, 32 - N_RANKS))
    pat80 = jnp.tile(examf, PAT // N_RANKS)
    y, exam_o, rel_o = _pbm_kernel(xf, relf, exam32, pat80)
    return (
        y.reshape(BATCH, N_RANKS),
        exam_o[:N_RANKS],
        rel_o.reshape(BATCH, N_RANKS),
    )


# speedup vs baseline: 1.1579x; 1.1152x over previous
import jax
import jax.numpy as jnp
from jax import lax
from jax.experimental import pallas as pl
from jax.experimental.pallas import tpu as pltpu
from jax.experimental.pallas import tpu_sc as plsc
from jax._src.pallas import mpmd
from jax._src.pallas.mosaic import core as _tpu_core

N_DOCS = 1000000
N_RANKS = 20
BATCH = 16384
TOTAL = BATCH * N_RANKS
NW = 32
B_PER_W = TOTAL // NW
PAT = 80
PAT_VECS = PAT // 16
CHUNKS = 4
CH_ELEMS = B_PER_W // CHUNKS
NSUB = 16

_vmesh = plsc.VectorSubcoreMesh(core_axis_name="c", subcore_axis_name="s")
_smesh = plsc.ScalarSubcoreMesh(axis_name="c", num_cores=2)


def _sigmoid16(v):
    return 1.0 / (1.0 + jnp.exp(-v))


def _scs_fn(x_hbm, rel_hbm, exam_hbm, pat_hbm, y_hbm, exam_out_hbm,
            rel_out_hbm, table_sh, done_sem):
    def inner(dsem):
        pltpu.async_copy(rel_hbm, table_sh, dsem).wait()
        for s in range(NSUB):
            pl.semaphore_signal(done_sem, 1, device_id={"s": s})

    pl.run_scoped(inner, pltpu.SemaphoreType.DMA)


def _tec_fn(x_hbm, rel_hbm, exam_hbm, pat_hbm, y_hbm, exam_out_hbm,
            rel_out_hbm, table_sh, done_sem):
    def inner(idx_v, rel_v, y_v, exam_v, pat_v, g0, g1, g2, g3,
              wsem_r, wsem_y):
        sid = lax.axis_index("s")
        wid = sid * 2 + lax.axis_index("c")
        base = wid * B_PER_W
        gsems = (g0, g1, g2, g3)

        pltpu.sync_copy(x_hbm.at[pl.ds(base, B_PER_W)], idx_v)

        pltpu.sync_copy(exam_hbm, exam_v)
        for v in range(2):
            sl = pl.ds(v * 16, 16)
            exam_v[sl] = _sigmoid16(exam_v[sl])
        pltpu.sync_copy(pat_hbm, pat_v)
        for v in range(PAT_VECS):
            sl = pl.ds(v * 16, 16)
            pat_v[sl] = _sigmoid16(pat_v[sl])
        pats = [pat_v[pl.ds(k * 16, 16)] for k in range(PAT_VECS)]

        pl.semaphore_wait(done_sem, 1)
        gathers = []
        for c in range(CHUNKS):
            sl = pl.ds(c * CH_ELEMS, CH_ELEMS)
            gathers.append(
                pltpu.async_copy(table_sh.at[idx_v.at[sl]], rel_v.at[sl],
                                 gsems[c]))

        writes = []
        for c in range(CHUNKS):
            gathers[c].wait()
            cbase = c * CH_ELEMS

            def body(j, _, cbase=cbase):
                b = cbase + j * PAT
                for k in range(PAT_VECS):
                    sl = pl.ds(b + k * 16, 16)
                    s = _sigmoid16(rel_v[sl])
                    rel_v[sl] = s
                    y_v[sl] = s * pats[k]
                return _

            lax.fori_loop(0, CH_ELEMS // PAT, body, None)

            sl = pl.ds(cbase, CH_ELEMS)
            osl = pl.ds(base + cbase, CH_ELEMS)
            writes.append(
                pltpu.async_copy(rel_v.at[sl], rel_out_hbm.at[osl], wsem_r))
            writes.append(
                pltpu.async_copy(y_v.at[sl], y_hbm.at[osl], wsem_y))

        @pl.when(wid == 0)
        def _():
            pltpu.sync_copy(exam_v, exam_out_hbm)

        for w in writes:
            w.wait()

    pl.run_scoped(
        inner,
        pltpu.VMEM((B_PER_W,), jnp.int32),
        pltpu.VMEM((B_PER_W,), jnp.float32),
        pltpu.VMEM((B_PER_W,), jnp.float32),
        pltpu.VMEM((32,), jnp.float32),
        pltpu.VMEM((PAT,), jnp.float32),
        pltpu.SemaphoreType.DMA,
        pltpu.SemaphoreType.DMA,
        pltpu.SemaphoreType.DMA,
        pltpu.SemaphoreType.DMA,
        pltpu.SemaphoreType.DMA,
        pltpu.SemaphoreType.DMA,
    )


_pbm_kernel = mpmd.mpmd_map(
    [(_smesh, _scs_fn), (_vmesh, _tec_fn)],
    out_types=[
        jax.ShapeDtypeStruct((TOTAL,), jnp.float32),
        jax.ShapeDtypeStruct((32,), jnp.float32),
        jax.ShapeDtypeStruct((TOTAL,), jnp.float32),
    ],
    scratch_types=[
        pltpu.VMEM_SHARED((N_DOCS,), jnp.float32),
        _tpu_core.SemaphoreType.REGULAR @ _vmesh,
    ],
)


@jax.jit
def kernel(x, rel_table, exam_table):
    xf = x.reshape(TOTAL)
    relf = rel_table.reshape(N_DOCS)
    examf = exam_table.reshape(N_RANKS)
    exam32 = jnp.pad(examf, (0, 32 - N_RANKS))
    pat80 = jnp.tile(examf, PAT // N_RANKS)
    y, exam_o, rel_o = _pbm_kernel(xf, relf, exam32, pat80)
    return (
        y.reshape(BATCH, N_RANKS),
        exam_o[:N_RANKS],
        rel_o.reshape(BATCH, N_RANKS),
    )


# speedup vs baseline: 1.1840x; 1.0226x over previous
import jax
import jax.numpy as jnp
from jax import lax
from jax.experimental import pallas as pl
from jax.experimental.pallas import tpu as pltpu
from jax.experimental.pallas import tpu_sc as plsc
from jax._src.pallas import mpmd
from jax._src.pallas.mosaic import core as _tpu_core

N_DOCS = 1000000
N_DOCS_PAD = 1000064
N_RANKS = 20
BATCH = 16384
TOTAL = BATCH * N_RANKS
NW = 32
B_PER_W = TOTAL // NW
PAT = 80
PAT_VECS = PAT // 16
CHUNKS = 4
CH_ELEMS = B_PER_W // CHUNKS
NSUB = 16

_vmesh = plsc.VectorSubcoreMesh(core_axis_name="c", subcore_axis_name="s")
_smesh = plsc.ScalarSubcoreMesh(axis_name="c", num_cores=2)


def _sigmoid16(v):
    return 1.0 / (1.0 + jnp.exp(-v))


def _scs_fn(x_hbm, rel_hbm, exam_hbm, y_hbm, exam_out_hbm,
            rel_out_hbm, table_sh, done_sem):
    def inner(dsem):
        pltpu.async_copy(rel_hbm, table_sh, dsem).wait()
        for s in range(NSUB):
            pl.semaphore_signal(done_sem, 1, device_id={"s": s})

    pl.run_scoped(inner, pltpu.SemaphoreType.DMA)


def _tec_fn(x_hbm, rel_hbm, exam_hbm, y_hbm, exam_out_hbm,
            rel_out_hbm, table_sh, done_sem):
    def inner(idx_v, rel_v, y_v, exam_v, exam2_v, g0, g1, g2, g3,
              wsem_r, wsem_y):
        sid = lax.axis_index("s")
        wid = sid * 2 + lax.axis_index("c")
        base = wid * B_PER_W
        gsems = (g0, g1, g2, g3)

        pltpu.sync_copy(x_hbm.at[pl.ds(base, B_PER_W)], idx_v)

        pltpu.sync_copy(exam_hbm, exam_v)
        e0 = _sigmoid16(exam_v[pl.ds(0, 16)])
        e1 = _sigmoid16(exam_v[pl.ds(4, 16)])
        exam2_v[pl.ds(0, 16)] = e0
        exam2_v[pl.ds(4, 16)] = e1
        exam2_v[pl.ds(20, 16)] = e0
        exam2_v[pl.ds(24, 16)] = e1
        pats = [exam2_v[pl.ds((16 * k) % N_RANKS, 16)]
                for k in range(PAT_VECS)]

        pl.semaphore_wait(done_sem, 1)
        gathers = []
        for c in range(CHUNKS):
            sl = pl.ds(c * CH_ELEMS, CH_ELEMS)
            gathers.append(
                pltpu.async_copy(table_sh.at[idx_v.at[sl]], rel_v.at[sl],
                                 gsems[c]))

        writes = []
        for c in range(CHUNKS):
            gathers[c].wait()
            cbase = c * CH_ELEMS

            def body(j, _, cbase=cbase):
                b = cbase + j * PAT
                for k in range(PAT_VECS):
                    sl = pl.ds(b + k * 16, 16)
                    s = _sigmoid16(rel_v[sl])
                    rel_v[sl] = s
                    y_v[sl] = s * pats[k]
                return _

            lax.fori_loop(0, CH_ELEMS // PAT, body, None)

            sl = pl.ds(cbase, CH_ELEMS)
            osl = pl.ds(base + cbase, CH_ELEMS)
            writes.append(
                pltpu.async_copy(rel_v.at[sl], rel_out_hbm.at[osl], wsem_r))
            writes.append(
                pltpu.async_copy(y_v.at[sl], y_hbm.at[osl], wsem_y))

        @pl.when(wid == 0)
        def _():
            pltpu.sync_copy(exam2_v.at[pl.ds(0, N_RANKS)], exam_out_hbm)

        for w in writes:
            w.wait()

    pl.run_scoped(
        inner,
        pltpu.VMEM((B_PER_W,), jnp.int32),
        pltpu.VMEM((B_PER_W,), jnp.float32),
        pltpu.VMEM((B_PER_W,), jnp.float32),
        pltpu.VMEM((N_RANKS,), jnp.float32),
        pltpu.VMEM((2 * N_RANKS,), jnp.float32),
        pltpu.SemaphoreType.DMA,
        pltpu.SemaphoreType.DMA,
        pltpu.SemaphoreType.DMA,
        pltpu.SemaphoreType.DMA,
        pltpu.SemaphoreType.DMA,
        pltpu.SemaphoreType.DMA,
    )


_pbm_kernel = mpmd.mpmd_map(
    [(_smesh, _scs_fn), (_vmesh, _tec_fn)],
    out_types=[
        jax.ShapeDtypeStruct((TOTAL,), jnp.float32),
        jax.ShapeDtypeStruct((N_RANKS,), jnp.float32),
        jax.ShapeDtypeStruct((TOTAL,), jnp.float32),
    ],
    scratch_types=[
        pltpu.VMEM_SHARED((N_DOCS_PAD,), jnp.float32),
        _tpu_core.SemaphoreType.REGULAR @ _vmesh,
    ],
    compiler_params=pltpu.CompilerParams(use_tc_tiling_on_sc=False),
)


@jax.jit
def kernel(x, rel_table, exam_table):
    xf = x.reshape(TOTAL)
    relf = jnp.pad(rel_table, ((0, N_DOCS_PAD - N_DOCS), (0, 0))).reshape(
        N_DOCS_PAD)
    examf = exam_table.reshape(N_RANKS)
    y, exam_o, rel_o = _pbm_kernel(xf, relf, examf)
    return (
        y.reshape(BATCH, N_RANKS),
        exam_o,
        rel_o.reshape(BATCH, N_RANKS),
    )


# speedup vs baseline: 1.8473x; 1.5602x over previous
import jax
import jax.numpy as jnp
from jax import lax
from jax.experimental import pallas as pl
from jax.experimental.pallas import tpu as pltpu
from jax.experimental.pallas import tpu_sc as plsc
from jax._src.pallas import mpmd
from jax._src.pallas.mosaic import core as _tpu_core

N_DOCS = 1000000
N_DOCS_PAD = 1000064
N_RANKS = 20
BATCH = 16384
TOTAL = BATCH * N_RANKS
NW = 32
B_PER_W = TOTAL // NW
PAT = 80
PAT_VECS = PAT // 16
CHUNKS = 4
CH_ELEMS = B_PER_W // CHUNKS
NSUB = 16

_vmesh = plsc.VectorSubcoreMesh(core_axis_name="c", subcore_axis_name="s")
_smesh = plsc.ScalarSubcoreMesh(axis_name="c", num_cores=2)


def _sigmoid16(v):
    return 1.0 / (1.0 + jnp.exp(-v))


def _scs_fn(x_hbm, rel_hbm, exam_hbm, y_hbm, exam_out_hbm,
            rel_out_hbm, table_sh, done_sem):
    def inner(dsem):
        pltpu.async_copy(rel_hbm, table_sh, dsem).wait()
        for s in range(NSUB):
            pl.semaphore_signal(done_sem, 1, device_id={"s": s})

    pl.run_scoped(inner, pltpu.SemaphoreType.DMA)


def _tec_fn(x_hbm, rel_hbm, exam_hbm, y_hbm, exam_out_hbm,
            rel_out_hbm, table_sh, done_sem):
    def inner(idx_v, rel_v, y_v, exam_v, exam2_v, g0, g1, g2, g3,
              wsem_r, wsem_y):
        sid = lax.axis_index("s")
        wid = sid * 2 + lax.axis_index("c")
        base = wid * B_PER_W
        gsems = (g0, g1, g2, g3)

        pltpu.sync_copy(x_hbm.at[pl.ds(base, B_PER_W)], idx_v)

        pltpu.sync_copy(exam_hbm, exam_v)
        e0 = _sigmoid16(exam_v[pl.ds(0, 16)])
        e1 = _sigmoid16(exam_v[pl.ds(4, 16)])
        exam2_v[pl.ds(0, 16)] = e0
        exam2_v[pl.ds(4, 16)] = e1

        def _splat_exam(c):
            ci = jnp.full((16,), c, jnp.int32)
            return jnp.where(
                c < 16,
                e0.at[ci].get(mode="promise_in_bounds"),
                e1.at[jnp.maximum(ci - 4, 0)].get(mode="promise_in_bounds"),
            )

        ca = base // BATCH
        ev_a = _splat_exam(ca)
        ev_b = _splat_exam(jnp.minimum(ca + 1, N_RANKS - 1))
        fb = (ca + 1) * BATCH
        iota16 = lax.iota(jnp.int32, 16)

        pl.semaphore_wait(done_sem, 1)
        gathers = []
        for c in range(CHUNKS):
            sl = pl.ds(c * CH_ELEMS, CH_ELEMS)
            gathers.append(
                pltpu.async_copy(table_sh.at[idx_v.at[sl]], rel_v.at[sl],
                                 gsems[c]))

        writes = []
        for c in range(CHUNKS):
            gathers[c].wait()
            cbase = c * CH_ELEMS

            def body(j, _, cbase=cbase):
                b = cbase + j * PAT
                for k in range(PAT_VECS):
                    sl = pl.ds(b + k * 16, 16)
                    s = _sigmoid16(rel_v[sl])
                    rel_v[sl] = s
                    p = (base + b + k * 16) + iota16
                    y_v[sl] = s * jnp.where(p < fb, ev_a, ev_b)
                return _

            lax.fori_loop(0, CH_ELEMS // PAT, body, None)

            sl = pl.ds(cbase, CH_ELEMS)
            osl = pl.ds(base + cbase, CH_ELEMS)
            writes.append(
                pltpu.async_copy(rel_v.at[sl], rel_out_hbm.at[osl], wsem_r))
            writes.append(
                pltpu.async_copy(y_v.at[sl], y_hbm.at[osl], wsem_y))

        @pl.when(wid == 0)
        def _():
            pltpu.sync_copy(exam2_v.at[pl.ds(0, N_RANKS)], exam_out_hbm)

        for w in writes:
            w.wait()

    pl.run_scoped(
        inner,
        pltpu.VMEM((B_PER_W,), jnp.int32),
        pltpu.VMEM((B_PER_W,), jnp.float32),
        pltpu.VMEM((B_PER_W,), jnp.float32),
        pltpu.VMEM((N_RANKS,), jnp.float32),
        pltpu.VMEM((2 * N_RANKS,), jnp.float32),
        pltpu.SemaphoreType.DMA,
        pltpu.SemaphoreType.DMA,
        pltpu.SemaphoreType.DMA,
        pltpu.SemaphoreType.DMA,
        pltpu.SemaphoreType.DMA,
        pltpu.SemaphoreType.DMA,
    )


_pbm_kernel = mpmd.mpmd_map(
    [(_smesh, _scs_fn), (_vmesh, _tec_fn)],
    out_types=[
        jax.ShapeDtypeStruct((TOTAL,), jnp.float32),
        jax.ShapeDtypeStruct((N_RANKS,), jnp.float32),
        jax.ShapeDtypeStruct((TOTAL,), jnp.float32),
    ],
    scratch_types=[
        pltpu.VMEM_SHARED((N_DOCS_PAD,), jnp.float32),
        _tpu_core.SemaphoreType.REGULAR @ _vmesh,
    ],
    compiler_params=pltpu.CompilerParams(use_tc_tiling_on_sc=False),
)


@jax.jit
def kernel(x, rel_table, exam_table):
    xf = x.T.reshape(TOTAL)
    relf = jnp.pad(rel_table, ((0, N_DOCS_PAD - N_DOCS), (0, 0))).reshape(
        N_DOCS_PAD)
    examf = exam_table.reshape(N_RANKS)
    y, exam_o, rel_o = _pbm_kernel(xf, relf, examf)
    return (
        y.reshape(N_RANKS, BATCH).T,
        exam_o,
        rel_o.reshape(N_RANKS, BATCH).T,
    )
